# baseline (speedup 1.0000x reference)
import functools
import math

import jax
import jax.numpy as jnp
from jax import lax
from jax.experimental import pallas as pl
from jax.experimental.pallas import tpu as pltpu
from jax.experimental.pallas import tpu_sc as plsc

SEQ_LEN = 8192
DIM = 64
VOCAB = 1000000
_NC, _NS = 2, 16
_NW = _NC * _NS
_B_PER_W = SEQ_LEN // _NW
_TBLOCK = 16384


def _tc_transpose_pairs(x_ref, o_ref):
    xt = x_ref[...].T
    x3 = xt.reshape(_TBLOCK // 32, 32, DIM)
    quarters = [
        x3[:, 8 * h:8 * h + 8, :].reshape(_TBLOCK // 4, DIM) for h in range(4)
    ]
    o_ref[...] = jnp.concatenate(quarters, axis=1).astype(jnp.bfloat16)


_mesh = plsc.VectorSubcoreMesh(core_axis_name="c", subcore_axis_name="s")


@functools.partial(
    pl.kernel,
    mesh=_mesh,
    out_type=jax.ShapeDtypeStruct((SEQ_LEN, 2 * DIM), jnp.int32),
    scratch_types=[
        pltpu.VMEM((_B_PER_W,), jnp.int32),
        pltpu.VMEM((_B_PER_W, 2 * DIM), jnp.int32),
        pltpu.SemaphoreType.DMA,
    ],
)
def _sc_gather(table_hbm, idx_hbm, out_hbm, idx_v, rows_v, sem):
    wid = lax.axis_index("s") * _NC + lax.axis_index("c")
    base = wid * _B_PER_W
    pltpu.sync_copy(idx_hbm.at[pl.ds(base, _B_PER_W)], idx_v)
    pltpu.async_copy(table_hbm.at[idx_v], rows_v, sem).wait()
    pltpu.sync_copy(rows_v, out_hbm.at[pl.ds(base, _B_PER_W)])


def _tc_select_add_t(x_ref, idx_ref, o_ref):
    h = (idx_ref[...] >> 3) & 3
    s = [x_ref[:, DIM * k:DIM * (k + 1)] for k in range(4)]
    rows = jnp.where(
        h == 0, s[0], jnp.where(h == 1, s[1], jnp.where(h == 2, s[2], s[3]))
    ).astype(jnp.float32)
    d_i = lax.broadcasted_iota(jnp.int32, (DIM, SEQ_LEN), 0)
    pair = (d_i // 2).astype(jnp.float32)
    pos = lax.broadcasted_iota(jnp.int32, (DIM, SEQ_LEN), 1).astype(jnp.float32)
    ang = pos * jnp.exp(pair * (-2.0 * math.log(10000.0) / DIM))
    iseven = d_i % 2 == 0
    o_ref[...] = rows.T + jnp.where(iseven, jnp.sin(ang), jnp.cos(ang))


def kernel(indices, table):
    idx = indices.astype(jnp.int32)
    pidx = ((idx >> 5) << 3) | (idx & 7)
    table_quads = pl.pallas_call(
        _tc_transpose_pairs,
        grid=(pl.cdiv(VOCAB, _TBLOCK),),
        in_specs=[pl.BlockSpec((DIM, _TBLOCK), lambda b: (0, b))],
        out_specs=pl.BlockSpec((_TBLOCK // 4, 4 * DIM), lambda b: (b, 0)),
        out_shape=jax.ShapeDtypeStruct((VOCAB // 4, 4 * DIM), jnp.bfloat16),
    )(table.T)
    tq_i32 = jax.lax.bitcast_convert_type(
        table_quads.reshape(VOCAB // 4, 2 * DIM, 2), jnp.int32
    )
    gathered = _sc_gather(tq_i32, pidx)
    g_bf16 = jax.lax.bitcast_convert_type(gathered, jnp.bfloat16).reshape(
        SEQ_LEN, 4 * DIM
    )
    out_t = pl.pallas_call(
        _tc_select_add_t,
        out_shape=jax.ShapeDtypeStruct((DIM, SEQ_LEN), jnp.float32),
    )(g_bf16, idx[:, None])
    return out_t.T[None, :, :]


# speedup vs baseline: 5.6916x; 5.6916x over previous
import functools
import math

import jax
import jax.numpy as jnp
from jax import lax
from jax.experimental import pallas as pl
from jax.experimental.pallas import tpu as pltpu
from jax.experimental.pallas import tpu_sc as plsc

SEQ_LEN = 8192
DIM = 64
VOCAB = 1000000
_NC, _NS = 2, 16
_NW = _NC * _NS
_B_PER_W = SEQ_LEN // _NW
_TBLOCK = 32768


def _tc_transpose_pairs(x_ref, o_ref):
    xt = x_ref[...].T
    x3 = xt.reshape(_TBLOCK // 16, 16, DIM)
    a = x3[:, :8, :].reshape(_TBLOCK // 2, DIM)
    b = x3[:, 8:, :].reshape(_TBLOCK // 2, DIM)
    o_ref[...] = jnp.concatenate([a, b], axis=1)


_mesh = plsc.VectorSubcoreMesh(core_axis_name="c", subcore_axis_name="s")


@functools.partial(
    pl.kernel,
    mesh=_mesh,
    out_type=jax.ShapeDtypeStruct((SEQ_LEN, 2 * DIM), jnp.float32),
    scratch_types=[
        pltpu.VMEM((_B_PER_W,), jnp.int32),
        pltpu.VMEM((_B_PER_W, 2 * DIM), jnp.float32),
        pltpu.SemaphoreType.DMA,
    ],
)
def _sc_gather(table_hbm, idx_hbm, out_hbm, idx_v, rows_v, sem):
    wid = lax.axis_index("s") * _NC + lax.axis_index("c")
    base = wid * _B_PER_W
    pltpu.sync_copy(idx_hbm.at[pl.ds(base, _B_PER_W)], idx_v)
    pltpu.async_copy(table_hbm.at[idx_v], rows_v, sem).wait()
    pltpu.sync_copy(rows_v, out_hbm.at[pl.ds(base, _B_PER_W)])


def _tc_select_add_t(x_ref, idx_ref, o_ref):
    half = ((idx_ref[...] >> 3) & 1) == 0
    rows = jnp.where(half, x_ref[:, :DIM], x_ref[:, DIM:])
    d_i = lax.broadcasted_iota(jnp.int32, (DIM, SEQ_LEN), 0)
    pair = (d_i // 2).astype(jnp.float32)
    pos = lax.broadcasted_iota(jnp.int32, (DIM, SEQ_LEN), 1).astype(jnp.float32)
    ang = pos * jnp.exp(pair * (-2.0 * math.log(10000.0) / DIM))
    iseven = d_i % 2 == 0
    o_ref[...] = rows.T + jnp.where(iseven, jnp.sin(ang), jnp.cos(ang))


def kernel(indices, table):
    idx = indices.astype(jnp.int32)
    pidx = ((idx >> 4) << 3) | (idx & 7)
    table_pairs = pl.pallas_call(
        _tc_transpose_pairs,
        grid=(pl.cdiv(VOCAB, _TBLOCK),),
        in_specs=[pl.BlockSpec((DIM, _TBLOCK), lambda b: (0, b))],
        out_specs=pl.BlockSpec((_TBLOCK // 2, 2 * DIM), lambda b: (b, 0)),
        out_shape=jax.ShapeDtypeStruct((VOCAB // 2, 2 * DIM), jnp.float32),
    )(table.T)
    gathered = _sc_gather(table_pairs, pidx)
    out_t = pl.pallas_call(
        _tc_select_add_t,
        out_shape=jax.ShapeDtypeStruct((DIM, SEQ_LEN), jnp.float32),
    )(gathered, idx[:, None])
    return out_t.T[None, :, :]
